# Initial kernel scaffold; baseline (speedup 1.0000x reference)
import jax
import jax.numpy as jnp
from jax.experimental import pallas as pl


def kernel(funcname_1, funcname_2, emb_table, W, U, b):
    raise NotImplementedError("write your pallas kernel here")


# speedup vs baseline: 1.2629x; 1.2629x over previous
import functools

import jax
import jax.numpy as jnp
from jax import lax
from jax.experimental import pallas as pl
from jax.experimental.pallas import tpu as pltpu
from jax.experimental.pallas import tpu_sc as plsc

B = 4096
L = 50
EMB = 64
FEAT = 64
NW = 32
ROWS = 2 * B * L
ROWS_PER_W = ROWS // NW
CHUNK = 128
GROUP = 4
CHUNKS_PER_W = ROWS_PER_W // CHUNK
GROUPS_PER_W = CHUNKS_PER_W // GROUP
GROUP_ROWS = GROUP * CHUNK


def _sc_gather_body(idx_hbm, table_hbm, out_hbm, idx_v, rows_v, gsem):
    c = lax.axis_index("c")
    s = lax.axis_index("s")
    wid = s * 2 + c
    pltpu.sync_copy(idx_hbm.at[wid], idx_v)
    base = wid * ROWS_PER_W

    def group(g, carry):
        descs = []
        for bi in range(GROUP):
            cidx = g * GROUP + bi
            d = pltpu.async_copy(
                table_hbm.at[idx_v.at[cidx]],
                rows_v.at[pl.ds(bi * CHUNK, CHUNK)],
                gsem,
            )
            descs.append(d)
        for d in descs:
            d.wait()
        pltpu.sync_copy(
            rows_v, out_hbm.at[pl.ds(base + g * GROUP_ROWS, GROUP_ROWS)]
        )
        return carry

    lax.fori_loop(0, GROUPS_PER_W, group, 0)


def _sc_gather(idx_grouped, table):
    mesh = plsc.VectorSubcoreMesh(core_axis_name="c", subcore_axis_name="s")
    f = pl.kernel(
        _sc_gather_body,
        out_type=jax.ShapeDtypeStruct((ROWS, EMB), jnp.float32),
        mesh=mesh,
        scratch_types=[
            pltpu.VMEM((CHUNKS_PER_W, CHUNK), jnp.int32),
            pltpu.VMEM((GROUP_ROWS, EMB), jnp.float32),
            pltpu.SemaphoreType.DMA,
        ],
        compiler_params=pltpu.CompilerParams(use_tc_tiling_on_sc=False),
    )
    return f(idx_grouped, table)


def _tc_rnn_body(x_ref, idx_ref, w_ref, u_ref, b_ref,
                 s1_ref, s2_ref, sim_ref, h_s):
    t = pl.program_id(0)

    @pl.when(t == 0)
    def _():
        h_s[...] = jnp.zeros_like(h_s)

    h = h_s[...]
    x = x_ref[0]
    xw = jnp.dot(x, w_ref[...], preferred_element_type=jnp.float32)
    hu = jnp.dot(h, u_ref[...], preferred_element_type=jnp.float32)
    h_new = jnp.tanh(xw + hu + b_ref[...])
    m = idx_ref[0] != 0
    h = jnp.where(m, h_new, h)
    h_s[...] = h

    @pl.when(t == L - 1)
    def _():
        s1 = h[:B]
        s2 = h[B:]
        n1 = jnp.sqrt(jnp.sum(s1 * s1, axis=1, keepdims=True)) + 1e-12
        n2 = jnp.sqrt(jnp.sum(s2 * s2, axis=1, keepdims=True)) + 1e-12
        s1_ref[...] = s1
        s2_ref[...] = s2
        sim_ref[...] = jnp.sum(s1 * s2, axis=1, keepdims=True) / (n1 * n2)


def _tc_rnn(x, idx_t, W, U, b):
    grid = (L,)
    return pl.pallas_call(
        _tc_rnn_body,
        grid=grid,
        in_specs=[
            pl.BlockSpec((1, 2 * B, EMB), lambda t: (t, 0, 0)),
            pl.BlockSpec((1, 2 * B, 1), lambda t: (t, 0, 0)),
            pl.BlockSpec((EMB, FEAT), lambda t: (0, 0)),
            pl.BlockSpec((FEAT, FEAT), lambda t: (0, 0)),
            pl.BlockSpec((1, FEAT), lambda t: (0, 0)),
        ],
        out_specs=[
            pl.BlockSpec((B, FEAT), lambda t: (0, 0)),
            pl.BlockSpec((B, FEAT), lambda t: (0, 0)),
            pl.BlockSpec((B, 1), lambda t: (0, 0)),
        ],
        out_shape=[
            jax.ShapeDtypeStruct((B, FEAT), jnp.float32),
            jax.ShapeDtypeStruct((B, FEAT), jnp.float32),
            jax.ShapeDtypeStruct((B, 1), jnp.float32),
        ],
        scratch_shapes=[pltpu.VMEM((2 * B, FEAT), jnp.float32)],
    )(x, idx_t, W, U, b)


@jax.jit
def kernel(funcname_1, funcname_2, emb_table, W, U, b):
    idx_cat = jnp.concatenate([funcname_1, funcname_2], axis=0)
    idx_t = idx_cat.T
    idx_flat = idx_t.reshape(NW, CHUNKS_PER_W, CHUNK)
    x = _sc_gather(idx_flat, emb_table)
    x = x.reshape(L, 2 * B, EMB)
    s1, s2, sim = _tc_rnn(x, idx_t.reshape(L, 2 * B, 1), W, U,
                          b.reshape(1, FEAT))
    return (s1, s2, sim.reshape(B))
